# baseline (speedup 1.0000x reference)
import functools

import jax
import jax.numpy as jnp
from jax import lax
from jax.experimental import pallas as pl
from jax.experimental.pallas import tpu as pltpu
from jax.experimental.pallas import tpu_sc as plsc

B = 4096
D = 64
V = 100000
ALPHA = 0.1

_INFO = plsc.get_sparse_core_info()
_NC = _INFO.num_cores
_NS = _INFO.num_subcores
_NW = _NC * _NS
_BPW = B // _NW
_CH = 32


def _gather_body(user_idx, item_idx, u_tbl, i_tbl, i_bias,
                 user_out, item_out, bias_out,
                 idx_u, idx_i, blk, row, buf, sel, bias_v, sem):
    wid = lax.axis_index("s") * _NC + lax.axis_index("c")
    base = wid * _BPW
    pltpu.sync_copy(user_idx.at[pl.ds(base, _BPW)], idx_u)
    pltpu.sync_copy(item_idx.at[pl.ds(base, _BPW)], idx_i)
    cb = pltpu.async_copy(i_bias.at[idx_i], bias_v, sem)

    def select_rows(ch_base):
        def body(e, _):
            r = row[pl.ds(ch_base + e, 16)][0]
            j = (r >> 1) & 7
            h = (r & 1) * 64
            for c in range(D // 16):
                sel[e, pl.ds(16 * c, 16)] = buf[e, j, pl.ds(h + 16 * c, 16)]
            return 0
        lax.fori_loop(0, _CH, body, 0)

    for tbl, idx, out in ((u_tbl, idx_u, user_out), (i_tbl, idx_i, item_out)):
        for g in range(_BPW // 16):
            v = idx[pl.ds(16 * g, 16)]
            blk[pl.ds(16 * g, 16)] = v >> 4
            row[pl.ds(16 * g, 16)] = v & 15
        for ch in range(_BPW // _CH):
            for g in range(_CH // 16):
                pltpu.async_copy(
                    tbl.at[blk.at[pl.ds(ch * _CH + g * 16, 16)]],
                    buf.at[pl.ds(g * 16, 16)], sem)
            pltpu.make_async_copy(tbl.at[pl.ds(0, _CH)], buf, sem).wait()
            select_rows(ch * _CH)
            pltpu.sync_copy(sel, out.at[pl.ds(base + ch * _CH, _CH)])

    cb.wait()
    pltpu.sync_copy(bias_v, bias_out.at[pl.ds(base, _BPW)])


_gather_call = functools.partial(
    pl.kernel,
    out_type=[
        jax.ShapeDtypeStruct((B, 2 * D), jnp.float32),
        jax.ShapeDtypeStruct((B, 2 * D), jnp.float32),
        jax.ShapeDtypeStruct((B,), jnp.float32),
    ],
    mesh=plsc.VectorSubcoreMesh(core_axis_name="c", subcore_axis_name="s"),
    compiler_params=pltpu.CompilerParams(use_tc_tiling_on_sc=True),
    scratch_types=[
        pltpu.VMEM((_BPW,), jnp.int32),
        pltpu.VMEM((_BPW,), jnp.int32),
        pltpu.VMEM((_BPW + 16,), jnp.int32),
        pltpu.VMEM((_BPW + 16,), jnp.int32),
        pltpu.VMEM((_CH, 8, 2 * D), jnp.float32),
        pltpu.VMEM((_CH, 2 * D), jnp.float32),
        pltpu.VMEM((_BPW,), jnp.float32),
        pltpu.SemaphoreType.DMA,
    ],
)(_gather_body)


def _loss_body(uv_ref, iv_ref, bias_ref, label_ref, out_ref):
    uv = uv_ref[:, :D]
    iv = iv_ref[:, :D]
    bias = bias_ref[...]
    t = label_ref[...]
    s = jnp.sum(iv, axis=0, keepdims=True)
    pre = jnp.sum(uv * s, axis=1) + bias
    bce = jnp.mean(jnp.maximum(pre, 0.0) - pre * t
                   + jnp.log1p(jnp.exp(-jnp.abs(pre))))
    reg = (jnp.sqrt(jnp.sum(iv * iv)) + jnp.sqrt(jnp.sum(uv * uv))
           + jnp.sqrt(jnp.sum(bias * bias)))
    out_ref[...] = jnp.broadcast_to(bce + ALPHA * reg, (1, 1))


def kernel(user, item, label, u_table, i_table, i_bias):
    u3 = u_table.reshape(V // 16, 8, 2 * D)
    i3 = i_table.reshape(V // 16, 8, 2 * D)
    user_v, item_v, bias_g = _gather_call(user, item, u3, i3, i_bias)
    loss = pl.pallas_call(
        _loss_body,
        out_shape=jax.ShapeDtypeStruct((1, 1), jnp.float32),
    )(user_v, item_v, bias_g, label)
    return loss[0, 0]


# speedup vs baseline: 1.1438x; 1.1438x over previous
import functools

import jax
import jax.numpy as jnp
from jax import lax
from jax.experimental import pallas as pl
from jax.experimental.pallas import tpu as pltpu
from jax.experimental.pallas import tpu_sc as plsc

B = 4096
D = 64
V = 100000
ALPHA = 0.1

_INFO = plsc.get_sparse_core_info()
_NC = _INFO.num_cores
_NS = _INFO.num_subcores
_NW = _NC * _NS
_BPW = B // _NW
_CH = 64


def _gather_body(user_idx, item_idx, u_tbl, i_tbl, i_bias,
                 user_out, item_out, bias_out,
                 idx_u, idx_i, urows, irows, bias_v,
                 s0, s1, s2, s3, sb, so):
    wid = lax.axis_index("s") * _NC + lax.axis_index("c")
    base = wid * _BPW
    pltpu.sync_copy(user_idx.at[pl.ds(base, _BPW)], idx_u)
    pltpu.sync_copy(item_idx.at[pl.ds(base, _BPW)], idx_i)
    gu0 = pltpu.async_copy(u_tbl.at[idx_u.at[pl.ds(0, _CH)]],
                           urows.at[pl.ds(0, _CH)], s0)
    gu1 = pltpu.async_copy(u_tbl.at[idx_u.at[pl.ds(_CH, _CH)]],
                           urows.at[pl.ds(_CH, _CH)], s1)
    gi0 = pltpu.async_copy(i_tbl.at[idx_i.at[pl.ds(0, _CH)]],
                           irows.at[pl.ds(0, _CH)], s2)
    gi1 = pltpu.async_copy(i_tbl.at[idx_i.at[pl.ds(_CH, _CH)]],
                           irows.at[pl.ds(_CH, _CH)], s3)
    cb = pltpu.async_copy(i_bias.at[idx_i], bias_v, sb)
    for h, rows, out, off in (
            (gu0, urows, user_out, 0), (gu1, urows, user_out, _CH),
            (gi0, irows, item_out, 0), (gi1, irows, item_out, _CH)):
        h.wait()
        pltpu.async_copy(rows.at[pl.ds(off, _CH)],
                         out.at[pl.ds(base + off, _CH)], so)
    cb.wait()
    pltpu.async_copy(bias_v, bias_out.at[pl.ds(base, _BPW)], so)
    for _ in range(4):
        pltpu.make_async_copy(u_tbl.at[pl.ds(0, _CH)],
                              urows.at[pl.ds(0, _CH)], so).wait()
    pltpu.make_async_copy(i_bias.at[pl.ds(0, _BPW)], bias_v, so).wait()


_gather_call = functools.partial(
    pl.kernel,
    out_type=[
        jax.ShapeDtypeStruct((B, D), jnp.float32),
        jax.ShapeDtypeStruct((B, D), jnp.float32),
        jax.ShapeDtypeStruct((B,), jnp.float32),
    ],
    mesh=plsc.VectorSubcoreMesh(core_axis_name="c", subcore_axis_name="s"),
    compiler_params=pltpu.CompilerParams(use_tc_tiling_on_sc=False),
    scratch_types=[
        pltpu.VMEM((_BPW,), jnp.int32),
        pltpu.VMEM((_BPW,), jnp.int32),
        pltpu.VMEM((_BPW, D), jnp.float32),
        pltpu.VMEM((_BPW, D), jnp.float32),
        pltpu.VMEM((_BPW,), jnp.float32),
        pltpu.SemaphoreType.DMA,
        pltpu.SemaphoreType.DMA,
        pltpu.SemaphoreType.DMA,
        pltpu.SemaphoreType.DMA,
        pltpu.SemaphoreType.DMA,
        pltpu.SemaphoreType.DMA,
    ],
)(_gather_body)


def _loss_body(uv_ref, iv_ref, bias_ref, label_ref, out_ref):
    uv = uv_ref[...]
    iv = iv_ref[...]
    bias = bias_ref[...]
    t = label_ref[...]
    s = jnp.sum(iv, axis=0, keepdims=True)
    pre = jnp.sum(uv * s, axis=1) + bias
    bce = jnp.mean(jnp.maximum(pre, 0.0) - pre * t
                   + jnp.log1p(jnp.exp(-jnp.abs(pre))))
    reg = (jnp.sqrt(jnp.sum(iv * iv)) + jnp.sqrt(jnp.sum(uv * uv))
           + jnp.sqrt(jnp.sum(bias * bias)))
    out_ref[...] = jnp.broadcast_to(bce + ALPHA * reg, (1, 1))


def kernel(user, item, label, u_table, i_table, i_bias):
    user_v, item_v, bias_g = _gather_call(user, item, u_table, i_table, i_bias)
    loss = pl.pallas_call(
        _loss_body,
        out_shape=jax.ShapeDtypeStruct((1, 1), jnp.float32),
    )(user_v, item_v, bias_g, label)
    return loss[0, 0]


# speedup vs baseline: 1.7108x; 1.4958x over previous
import functools

import jax
import jax.numpy as jnp
from jax import lax
from jax.experimental import pallas as pl
from jax.experimental.pallas import tpu as pltpu
from jax.experimental.pallas import tpu_sc as plsc

B = 4096
D = 64
V = 100000
ALPHA = 0.1

_INFO = plsc.get_sparse_core_info()
_NC = _INFO.num_cores
_NS = _INFO.num_subcores
_NW = _NC * _NS
_BPW = B // _NW
_CH = 64


def _gather_body(user_idx, item_idx, u_tbl, i_tbl, i_bias,
                 user_out, item_out, bias_out,
                 idx_u, idx_i, blk, row, buf, sel, bias_v, sem):
    wid = lax.axis_index("s") * _NC + lax.axis_index("c")
    base = wid * _BPW
    pltpu.sync_copy(user_idx.at[pl.ds(base, _BPW)], idx_u)
    pltpu.sync_copy(item_idx.at[pl.ds(base, _BPW)], idx_i)
    cb = pltpu.async_copy(i_bias.at[idx_i], bias_v, sem)

    def select_rows(ch_base):
        def body(e, _):
            r = row[pl.ds(ch_base + e, 16)][0]
            for c in range(D // 16):
                sel[e, pl.ds(16 * c, 16)] = buf[e, r, pl.ds(16 * c, 16)]
            return 0
        lax.fori_loop(0, _CH, body, 0)

    for tbl, idx, out in ((u_tbl, idx_u, user_out), (i_tbl, idx_i, item_out)):
        for g in range(_BPW // 16):
            v = idx[pl.ds(16 * g, 16)]
            blk[pl.ds(16 * g, 16)] = v >> 3
            row[pl.ds(16 * g, 16)] = v & 7
        for ch in range(_BPW // _CH):
            def fire(grp, _):
                vec = blk[pl.ds(ch * _CH + grp * 16, 16)]
                for j in range(16):
                    pltpu.async_copy(tbl.at[vec[j]],
                                     buf.at[grp * 16 + j], sem)
                return 0
            lax.fori_loop(0, _CH // 16, fire, 0)
            pltpu.make_async_copy(tbl.at[pl.ds(0, _CH)], buf, sem).wait()
            select_rows(ch * _CH)
            pltpu.sync_copy(sel, out.at[pl.ds(base + ch * _CH, _CH)])

    cb.wait()
    pltpu.sync_copy(bias_v, bias_out.at[pl.ds(base, _BPW)])


_gather_call = functools.partial(
    pl.kernel,
    out_type=[
        jax.ShapeDtypeStruct((B, 2 * D), jnp.float32),
        jax.ShapeDtypeStruct((B, 2 * D), jnp.float32),
        jax.ShapeDtypeStruct((B,), jnp.float32),
    ],
    mesh=plsc.VectorSubcoreMesh(core_axis_name="c", subcore_axis_name="s"),
    compiler_params=pltpu.CompilerParams(use_tc_tiling_on_sc=True),
    scratch_types=[
        pltpu.VMEM((_BPW,), jnp.int32),
        pltpu.VMEM((_BPW,), jnp.int32),
        pltpu.VMEM((_BPW + 16,), jnp.int32),
        pltpu.VMEM((_BPW + 16,), jnp.int32),
        pltpu.VMEM((_CH, 8, D), jnp.float32),
        pltpu.VMEM((_CH, 2 * D), jnp.float32),
        pltpu.VMEM((_BPW,), jnp.float32),
        pltpu.SemaphoreType.DMA,
    ],
)(_gather_body)


def _loss_body(uv_ref, iv_ref, bias_ref, label_ref, out_ref):
    uv = uv_ref[:, :D]
    iv = iv_ref[:, :D]
    bias = bias_ref[...]
    t = label_ref[...]
    s = jnp.sum(iv, axis=0, keepdims=True)
    pre = jnp.sum(uv * s, axis=1) + bias
    bce = jnp.mean(jnp.maximum(pre, 0.0) - pre * t
                   + jnp.log1p(jnp.exp(-jnp.abs(pre))))
    reg = (jnp.sqrt(jnp.sum(iv * iv)) + jnp.sqrt(jnp.sum(uv * uv))
           + jnp.sqrt(jnp.sum(bias * bias)))
    out_ref[...] = jnp.broadcast_to(bce + ALPHA * reg, (1, 1))


def kernel(user, item, label, u_table, i_table, i_bias):
    u3 = u_table.reshape(V // 8, 8, D)
    i3 = i_table.reshape(V // 8, 8, D)
    user_v, item_v, bias_g = _gather_call(user, item, u3, i3, i_bias)
    loss = pl.pallas_call(
        _loss_body,
        out_shape=jax.ShapeDtypeStruct((1, 1), jnp.float32),
    )(user_v, item_v, bias_g, label)
    return loss[0, 0]
